# Initial kernel scaffold; baseline (speedup 1.0000x reference)
import jax
import jax.numpy as jnp
from jax.experimental import pallas as pl


def kernel(descriptors0, descriptors1, keypoints0, keypoints1):
    raise NotImplementedError("write your pallas kernel here")


# speedup vs baseline: 53.4326x; 53.4326x over previous
import functools

import jax
import jax.numpy as jnp
from jax import lax
from jax.experimental import pallas as pl
from jax.experimental.pallas import tpu as pltpu
from jax.experimental.pallas import tpu_sc as plsc

SQRT_2 = 1.414213

_BN = 512
_BM = 512


def _top2_body(d0_ref, d1_ref, fv1, fi1, fv2, fok, bv1, bi1, bv2, bok):
    i = pl.program_id(0)
    j = pl.program_id(1)
    s = lax.dot_general(
        d0_ref[...], d1_ref[...],
        dimension_numbers=(((0,), (0,)), ((), ())),
        preferred_element_type=jnp.float32,
    )

    col_iota = lax.broadcasted_iota(jnp.int32, s.shape, 1)
    row_iota = lax.broadcasted_iota(jnp.int32, s.shape, 0)

    w1 = jnp.max(s, axis=1)
    a1 = jnp.argmax(s, axis=1).astype(jnp.int32)
    w2 = jnp.max(jnp.where(col_iota == a1[:, None], -jnp.inf, s), axis=1)
    w1 = w1[None, :]
    w2 = w2[None, :]
    g1 = (a1 + j * _BM)[None, :]

    first = j == 0
    cv1 = jnp.where(first, -3.0, fv1[pl.ds(i, 1), :])
    ci1 = jnp.where(first, 0, fi1[pl.ds(i, 1), :])
    cv2 = jnp.where(first, -3.0, fv2[pl.ds(i, 1), :])
    take = w1 > cv1
    m1 = jnp.where(take, w1, cv1)
    mi = jnp.where(take, g1, ci1)
    m2 = jnp.maximum(jnp.minimum(cv1, w1), jnp.maximum(cv2, w2))
    fv1[pl.ds(i, 1), :] = m1
    fi1[pl.ds(i, 1), :] = mi
    fv2[pl.ds(i, 1), :] = m2
    dist1 = SQRT_2 * jnp.sqrt(jnp.maximum(1.0 - m1, 1e-6))
    dist2 = SQRT_2 * jnp.sqrt(jnp.maximum(1.0 - m2, 1e-6))
    fok[pl.ds(i, 1), :] = (dist1 / dist2 < 1.0).astype(jnp.int32)

    u1 = jnp.max(s, axis=0)[None, :]
    b1 = jnp.argmax(s, axis=0).astype(jnp.int32)
    u2 = jnp.max(jnp.where(row_iota == b1[None, :], -jnp.inf, s), axis=0)[None, :]
    h1 = (b1 + i * _BN)[None, :]

    firstb = i == 0
    dv1 = jnp.where(firstb, -3.0, bv1[pl.ds(j, 1), :])
    di1 = jnp.where(firstb, 0, bi1[pl.ds(j, 1), :])
    dv2 = jnp.where(firstb, -3.0, bv2[pl.ds(j, 1), :])
    takeb = u1 > dv1
    n1 = jnp.where(takeb, u1, dv1)
    ni = jnp.where(takeb, h1, di1)
    n2 = jnp.maximum(jnp.minimum(dv1, u1), jnp.maximum(dv2, u2))
    bv1[pl.ds(j, 1), :] = n1
    bi1[pl.ds(j, 1), :] = ni
    bv2[pl.ds(j, 1), :] = n2
    bd1 = SQRT_2 * jnp.sqrt(jnp.maximum(1.0 - n1, 1e-6))
    bd2 = SQRT_2 * jnp.sqrt(jnp.maximum(1.0 - n2, 1e-6))
    bok[pl.ds(j, 1), :] = (bd1 / bd2 < 1.0).astype(jnp.int32)


def _run_top2(d0, d1):
    F, N = d0.shape
    _, M = d1.shape
    I, J = N // _BN, M // _BM
    grid = (I, J)
    blk_out = lambda rows, cols: pl.BlockSpec((rows, cols), lambda i, j: (0, 0))
    out_shapes = [
        jax.ShapeDtypeStruct((I, _BN), jnp.float32),
        jax.ShapeDtypeStruct((I, _BN), jnp.int32),
        jax.ShapeDtypeStruct((I, _BN), jnp.float32),
        jax.ShapeDtypeStruct((I, _BN), jnp.int32),
        jax.ShapeDtypeStruct((J, _BM), jnp.float32),
        jax.ShapeDtypeStruct((J, _BM), jnp.int32),
        jax.ShapeDtypeStruct((J, _BM), jnp.float32),
        jax.ShapeDtypeStruct((J, _BM), jnp.int32),
    ]
    out_specs = [
        blk_out(I, _BN), blk_out(I, _BN), blk_out(I, _BN), blk_out(I, _BN),
        blk_out(J, _BM), blk_out(J, _BM), blk_out(J, _BM), blk_out(J, _BM),
    ]
    outs = pl.pallas_call(
        _top2_body,
        grid=grid,
        in_specs=[
            pl.BlockSpec((F, _BN), lambda i, j: (0, i)),
            pl.BlockSpec((F, _BM), lambda i, j: (0, j)),
        ],
        out_specs=out_specs,
        out_shape=out_shapes,
        compiler_params=pltpu.CompilerParams(
            dimension_semantics=("arbitrary", "arbitrary"),
        ),
    )(d0, d1)
    _, fi1, _, fok, _, bi1, _, bok = outs
    return (fi1.reshape(N), fok.reshape(N), bi1.reshape(M), bok.reshape(M))


def _mutual_match_sc(fwd_best, fwd_ok, bck_best, bck_ok):
    N = fwd_best.shape[0]
    M = bck_best.shape[0]
    info = plsc.get_sparse_core_info()
    NC, NS, L = info.num_cores, info.num_subcores, info.num_lanes
    NW = NC * NS
    chunk = N // NW
    mesh = plsc.VectorSubcoreMesh(core_axis_name="c", subcore_axis_name="s")

    @functools.partial(
        pl.kernel,
        mesh=mesh,
        out_type=[
            jax.ShapeDtypeStruct((N,), jnp.int32),
            jax.ShapeDtypeStruct((N,), jnp.int32),
        ],
        scratch_types=[
            pltpu.VMEM((chunk,), jnp.int32),
            pltpu.VMEM((chunk,), jnp.int32),
            pltpu.VMEM((chunk,), jnp.int32),
            pltpu.VMEM((chunk,), jnp.int32),
            pltpu.VMEM((chunk,), jnp.int32),
            pltpu.VMEM((chunk,), jnp.int32),
            pltpu.SemaphoreType.DMA,
        ],
    )
    def body(fb_hbm, fo_hbm, bb_hbm, bo_hbm, idx_hbm, ms_hbm,
             fb_v, fo_v, gb_v, go_v, oi_v, os_v, sem):
        wid = lax.axis_index("s") * NC + lax.axis_index("c")
        base = wid * chunk
        pltpu.sync_copy(fb_hbm.at[pl.ds(base, chunk)], fb_v)
        pltpu.sync_copy(fo_hbm.at[pl.ds(base, chunk)], fo_v)
        pltpu.async_copy(bb_hbm.at[fb_v], gb_v, sem).wait()
        pltpu.async_copy(bo_hbm.at[fb_v], go_v, sem).wait()

        for t in range(chunk // L):
            off = t * L
            idx = fb_v[pl.ds(off, L)]
            f_ok = fo_v[pl.ds(off, L)]
            g_best = gb_v[pl.ds(off, L)]
            g_ok = go_v[pl.ds(off, L)]
            row = lax.iota(jnp.int32, L) + (base + off)
            ok = (f_ok > 0) & (g_ok > 0) & (g_best == row)
            res = jnp.where(ok, idx, -1)
            oi_v[pl.ds(off, L)] = res
            os_v[pl.ds(off, L)] = jnp.where(res > 0, 1, 0).astype(jnp.int32)

        pltpu.sync_copy(oi_v, idx_hbm.at[pl.ds(base, chunk)])
        pltpu.sync_copy(os_v, ms_hbm.at[pl.ds(base, chunk)])

    return body(fwd_best, fwd_ok, bck_best, bck_ok)


def kernel(descriptors0, descriptors1, keypoints0, keypoints1):
    d0 = descriptors0[0]
    d1 = descriptors1[0]
    N = d0.shape[1]
    M = d1.shape[1]
    fwd_best, fwd_ok, bck_best, bck_ok = _run_top2(d0, d1)
    indices0, mscores0 = _mutual_match_sc(fwd_best, fwd_ok, bck_best, bck_ok)
    indices0 = indices0[None, :]
    mscores0 = mscores0[None, :]
    matches1 = jnp.full((1, M), -1, dtype=jnp.int32)
    mscores1 = jnp.zeros((1, M), dtype=keypoints1.dtype)
    return (indices0, matches1, mscores0, mscores1)


# speedup vs baseline: 86.0027x; 1.6096x over previous
import functools

import jax
import jax.numpy as jnp
from jax import lax
from jax.experimental import pallas as pl
from jax.experimental.pallas import tpu as pltpu
from jax.experimental.pallas import tpu_sc as plsc

SQRT_2 = 1.414213

_BN = 512
_BM = 512


def _top2_body(d0_ref, d1_ref, fv1, fi1, fv2, fok, bv1, bi1, bv2, bok):
    i = pl.program_id(0)
    j = pl.program_id(1)
    s = lax.dot_general(
        d0_ref[...], d1_ref[...],
        dimension_numbers=(((0,), (0,)), ((), ())),
        preferred_element_type=jnp.float32,
    )

    col_iota = lax.broadcasted_iota(jnp.int32, s.shape, 1)
    row_iota = lax.broadcasted_iota(jnp.int32, s.shape, 0)

    w1 = jnp.max(s, axis=1, keepdims=True)
    eq = s == w1
    a1 = jnp.min(jnp.where(eq, col_iota, _BM), axis=1, keepdims=True) + j * _BM
    cnt = jnp.sum(eq.astype(jnp.int32), axis=1, keepdims=True)
    w2 = jnp.max(jnp.where(eq, -jnp.inf, s), axis=1, keepdims=True)
    w2 = jnp.where(cnt > 1, w1, w2)

    rs = pl.ds(i * _BN, _BN)
    first = j == 0
    cv1 = jnp.where(first, -3.0, fv1[rs, :])
    ci1 = jnp.where(first, 0, fi1[rs, :])
    cv2 = jnp.where(first, -3.0, fv2[rs, :])
    take = w1 > cv1
    m1 = jnp.where(take, w1, cv1)
    mi = jnp.where(take, a1, ci1)
    m2 = jnp.maximum(jnp.minimum(cv1, w1), jnp.maximum(cv2, w2))
    fv1[rs, :] = m1
    fi1[rs, :] = mi
    fv2[rs, :] = m2
    dist1 = SQRT_2 * jnp.sqrt(jnp.maximum(1.0 - m1, 1e-6))
    dist2 = SQRT_2 * jnp.sqrt(jnp.maximum(1.0 - m2, 1e-6))
    fok[rs, :] = (dist1 / dist2 < 1.0).astype(jnp.int32)

    u1 = jnp.max(s, axis=0, keepdims=True)
    eqb = s == u1
    b1 = jnp.min(jnp.where(eqb, row_iota, _BN), axis=0, keepdims=True) + i * _BN
    cntb = jnp.sum(eqb.astype(jnp.int32), axis=0, keepdims=True)
    u2 = jnp.max(jnp.where(eqb, -jnp.inf, s), axis=0, keepdims=True)
    u2 = jnp.where(cntb > 1, u1, u2)

    firstb = i == 0
    dv1 = jnp.where(firstb, -3.0, bv1[pl.ds(j, 1), :])
    di1 = jnp.where(firstb, 0, bi1[pl.ds(j, 1), :])
    dv2 = jnp.where(firstb, -3.0, bv2[pl.ds(j, 1), :])
    takeb = u1 > dv1
    n1 = jnp.where(takeb, u1, dv1)
    ni = jnp.where(takeb, b1, di1)
    n2 = jnp.maximum(jnp.minimum(dv1, u1), jnp.maximum(dv2, u2))
    bv1[pl.ds(j, 1), :] = n1
    bi1[pl.ds(j, 1), :] = ni
    bv2[pl.ds(j, 1), :] = n2
    bd1 = SQRT_2 * jnp.sqrt(jnp.maximum(1.0 - n1, 1e-6))
    bd2 = SQRT_2 * jnp.sqrt(jnp.maximum(1.0 - n2, 1e-6))
    bok[pl.ds(j, 1), :] = (bd1 / bd2 < 1.0).astype(jnp.int32)


def _run_top2(d0, d1):
    F, N = d0.shape
    _, M = d1.shape
    I, J = N // _BN, M // _BM
    grid = (I, J)
    blk_out = lambda rows, cols: pl.BlockSpec((rows, cols), lambda i, j: (0, 0))
    out_shapes = [
        jax.ShapeDtypeStruct((N, 1), jnp.float32),
        jax.ShapeDtypeStruct((N, 1), jnp.int32),
        jax.ShapeDtypeStruct((N, 1), jnp.float32),
        jax.ShapeDtypeStruct((N, 1), jnp.int32),
        jax.ShapeDtypeStruct((J, _BM), jnp.float32),
        jax.ShapeDtypeStruct((J, _BM), jnp.int32),
        jax.ShapeDtypeStruct((J, _BM), jnp.float32),
        jax.ShapeDtypeStruct((J, _BM), jnp.int32),
    ]
    out_specs = [
        blk_out(N, 1), blk_out(N, 1), blk_out(N, 1), blk_out(N, 1),
        blk_out(J, _BM), blk_out(J, _BM), blk_out(J, _BM), blk_out(J, _BM),
    ]
    outs = pl.pallas_call(
        _top2_body,
        grid=grid,
        in_specs=[
            pl.BlockSpec((F, _BN), lambda i, j: (0, i)),
            pl.BlockSpec((F, _BM), lambda i, j: (0, j)),
        ],
        out_specs=out_specs,
        out_shape=out_shapes,
        compiler_params=pltpu.CompilerParams(
            dimension_semantics=("arbitrary", "arbitrary"),
        ),
    )(d0, d1)
    _, fi1, _, fok, _, bi1, _, bok = outs
    return (fi1.reshape(N), fok.reshape(N), bi1.reshape(M), bok.reshape(M))


def _mutual_match_sc(fwd_best, fwd_ok, bck_best, bck_ok):
    N = fwd_best.shape[0]
    M = bck_best.shape[0]
    info = plsc.get_sparse_core_info()
    NC, NS, L = info.num_cores, info.num_subcores, info.num_lanes
    NW = NC * NS
    chunk = N // NW
    mesh = plsc.VectorSubcoreMesh(core_axis_name="c", subcore_axis_name="s")

    @functools.partial(
        pl.kernel,
        mesh=mesh,
        out_type=[
            jax.ShapeDtypeStruct((N,), jnp.int32),
            jax.ShapeDtypeStruct((N,), jnp.int32),
        ],
        scratch_types=[
            pltpu.VMEM((chunk,), jnp.int32),
            pltpu.VMEM((chunk,), jnp.int32),
            pltpu.VMEM((chunk,), jnp.int32),
            pltpu.VMEM((chunk,), jnp.int32),
            pltpu.VMEM((chunk,), jnp.int32),
            pltpu.VMEM((chunk,), jnp.int32),
            pltpu.SemaphoreType.DMA,
        ],
    )
    def body(fb_hbm, fo_hbm, bb_hbm, bo_hbm, idx_hbm, ms_hbm,
             fb_v, fo_v, gb_v, go_v, oi_v, os_v, sem):
        wid = lax.axis_index("s") * NC + lax.axis_index("c")
        base = wid * chunk
        pltpu.sync_copy(fb_hbm.at[pl.ds(base, chunk)], fb_v)
        pltpu.sync_copy(fo_hbm.at[pl.ds(base, chunk)], fo_v)
        pltpu.async_copy(bb_hbm.at[fb_v], gb_v, sem).wait()
        pltpu.async_copy(bo_hbm.at[fb_v], go_v, sem).wait()

        for t in range(chunk // L):
            off = t * L
            idx = fb_v[pl.ds(off, L)]
            f_ok = fo_v[pl.ds(off, L)]
            g_best = gb_v[pl.ds(off, L)]
            g_ok = go_v[pl.ds(off, L)]
            row = lax.iota(jnp.int32, L) + (base + off)
            ok = (f_ok > 0) & (g_ok > 0) & (g_best == row)
            res = jnp.where(ok, idx, -1)
            oi_v[pl.ds(off, L)] = res
            os_v[pl.ds(off, L)] = jnp.where(res > 0, 1, 0).astype(jnp.int32)

        pltpu.sync_copy(oi_v, idx_hbm.at[pl.ds(base, chunk)])
        pltpu.sync_copy(os_v, ms_hbm.at[pl.ds(base, chunk)])

    return body(fwd_best, fwd_ok, bck_best, bck_ok)


def kernel(descriptors0, descriptors1, keypoints0, keypoints1):
    d0 = descriptors0[0]
    d1 = descriptors1[0]
    N = d0.shape[1]
    M = d1.shape[1]
    fwd_best, fwd_ok, bck_best, bck_ok = _run_top2(d0, d1)
    indices0, mscores0 = _mutual_match_sc(fwd_best, fwd_ok, bck_best, bck_ok)
    indices0 = indices0[None, :]
    mscores0 = mscores0[None, :]
    matches1 = jnp.full((1, M), -1, dtype=jnp.int32)
    mscores1 = jnp.zeros((1, M), dtype=keypoints1.dtype)
    return (indices0, matches1, mscores0, mscores1)


# speedup vs baseline: 112.4561x; 1.3076x over previous
import functools

import jax
import jax.numpy as jnp
from jax import lax
from jax.experimental import pallas as pl
from jax.experimental.pallas import tpu as pltpu
from jax.experimental.pallas import tpu_sc as plsc

SQRT_2 = 1.414213

_BN = 512
_BM = 1024


def _top2_body(d0_ref, d1_ref, fv1, fi1, fv2, fok, bv1, bi1, bv2, bok):
    i = pl.program_id(0)
    j = pl.program_id(1)
    s = lax.dot_general(
        d0_ref[...], d1_ref[...],
        dimension_numbers=(((0,), (0,)), ((), ())),
        preferred_element_type=jnp.float32,
    )

    col_iota = lax.broadcasted_iota(jnp.int32, s.shape, 1)
    row_iota = lax.broadcasted_iota(jnp.int32, s.shape, 0)

    w1 = jnp.max(s, axis=1, keepdims=True)
    eq = s == w1
    a1 = jnp.min(jnp.where(eq, col_iota, _BM), axis=1, keepdims=True) + j * _BM
    cnt = jnp.sum(eq.astype(jnp.int32), axis=1, keepdims=True)
    w2 = jnp.max(jnp.where(eq, -jnp.inf, s), axis=1, keepdims=True)
    w2 = jnp.where(cnt > 1, w1, w2)

    rs = pl.ds(i * _BN, _BN)
    first = j == 0
    cv1 = jnp.where(first, -3.0, fv1[rs, :])
    ci1 = jnp.where(first, 0, fi1[rs, :])
    cv2 = jnp.where(first, -3.0, fv2[rs, :])
    take = w1 > cv1
    m1 = jnp.where(take, w1, cv1)
    mi = jnp.where(take, a1, ci1)
    m2 = jnp.maximum(jnp.minimum(cv1, w1), jnp.maximum(cv2, w2))
    fv1[rs, :] = m1
    fi1[rs, :] = mi
    fv2[rs, :] = m2

    @pl.when(j == pl.num_programs(1) - 1)
    def _fwd_epilogue():
        dist1 = SQRT_2 * jnp.sqrt(jnp.maximum(1.0 - m1, 1e-6))
        dist2 = SQRT_2 * jnp.sqrt(jnp.maximum(1.0 - m2, 1e-6))
        fok[rs, :] = (dist1 / dist2 < 1.0).astype(jnp.int32)

    u1 = jnp.max(s, axis=0, keepdims=True)
    eqb = s == u1
    b1 = jnp.min(jnp.where(eqb, row_iota, _BN), axis=0, keepdims=True) + i * _BN
    cntb = jnp.sum(eqb.astype(jnp.int32), axis=0, keepdims=True)
    u2 = jnp.max(jnp.where(eqb, -jnp.inf, s), axis=0, keepdims=True)
    u2 = jnp.where(cntb > 1, u1, u2)

    firstb = i == 0
    dv1 = jnp.where(firstb, -3.0, bv1[pl.ds(j, 1), :])
    di1 = jnp.where(firstb, 0, bi1[pl.ds(j, 1), :])
    dv2 = jnp.where(firstb, -3.0, bv2[pl.ds(j, 1), :])
    takeb = u1 > dv1
    n1 = jnp.where(takeb, u1, dv1)
    ni = jnp.where(takeb, b1, di1)
    n2 = jnp.maximum(jnp.minimum(dv1, u1), jnp.maximum(dv2, u2))
    bv1[pl.ds(j, 1), :] = n1
    bi1[pl.ds(j, 1), :] = ni
    bv2[pl.ds(j, 1), :] = n2

    @pl.when(i == pl.num_programs(0) - 1)
    def _bck_epilogue():
        bd1 = SQRT_2 * jnp.sqrt(jnp.maximum(1.0 - n1, 1e-6))
        bd2 = SQRT_2 * jnp.sqrt(jnp.maximum(1.0 - n2, 1e-6))
        bok[pl.ds(j, 1), :] = (bd1 / bd2 < 1.0).astype(jnp.int32)


def _run_top2(d0, d1):
    F, N = d0.shape
    _, M = d1.shape
    I, J = N // _BN, M // _BM
    grid = (I, J)
    blk_out = lambda rows, cols: pl.BlockSpec((rows, cols), lambda i, j: (0, 0))
    out_shapes = [
        jax.ShapeDtypeStruct((N, 1), jnp.float32),
        jax.ShapeDtypeStruct((N, 1), jnp.int32),
        jax.ShapeDtypeStruct((N, 1), jnp.float32),
        jax.ShapeDtypeStruct((N, 1), jnp.int32),
        jax.ShapeDtypeStruct((J, _BM), jnp.float32),
        jax.ShapeDtypeStruct((J, _BM), jnp.int32),
        jax.ShapeDtypeStruct((J, _BM), jnp.float32),
        jax.ShapeDtypeStruct((J, _BM), jnp.int32),
    ]
    out_specs = [
        blk_out(N, 1), blk_out(N, 1), blk_out(N, 1), blk_out(N, 1),
        blk_out(J, _BM), blk_out(J, _BM), blk_out(J, _BM), blk_out(J, _BM),
    ]
    outs = pl.pallas_call(
        _top2_body,
        grid=grid,
        in_specs=[
            pl.BlockSpec((F, _BN), lambda i, j: (0, i)),
            pl.BlockSpec((F, _BM), lambda i, j: (0, j)),
        ],
        out_specs=out_specs,
        out_shape=out_shapes,
        compiler_params=pltpu.CompilerParams(
            dimension_semantics=("arbitrary", "arbitrary"),
        ),
    )(d0, d1)
    _, fi1, _, fok, _, bi1, _, bok = outs
    return (fi1.reshape(N), fok.reshape(N), bi1.reshape(M), bok.reshape(M))


def _mutual_match_sc(fwd_best, fwd_ok, bck_best, bck_ok):
    N = fwd_best.shape[0]
    M = bck_best.shape[0]
    info = plsc.get_sparse_core_info()
    NC, NS, L = info.num_cores, info.num_subcores, info.num_lanes
    NW = NC * NS
    chunk = N // NW
    mesh = plsc.VectorSubcoreMesh(core_axis_name="c", subcore_axis_name="s")

    @functools.partial(
        pl.kernel,
        mesh=mesh,
        out_type=[
            jax.ShapeDtypeStruct((N,), jnp.int32),
            jax.ShapeDtypeStruct((N,), jnp.int32),
        ],
        scratch_types=[
            pltpu.VMEM((chunk,), jnp.int32),
            pltpu.VMEM((chunk,), jnp.int32),
            pltpu.VMEM((chunk,), jnp.int32),
            pltpu.VMEM((chunk,), jnp.int32),
            pltpu.VMEM((chunk,), jnp.int32),
            pltpu.VMEM((chunk,), jnp.int32),
            pltpu.SemaphoreType.DMA,
        ],
    )
    def body(fb_hbm, fo_hbm, bb_hbm, bo_hbm, idx_hbm, ms_hbm,
             fb_v, fo_v, gb_v, go_v, oi_v, os_v, sem):
        wid = lax.axis_index("s") * NC + lax.axis_index("c")
        base = wid * chunk
        pltpu.sync_copy(fb_hbm.at[pl.ds(base, chunk)], fb_v)
        pltpu.sync_copy(fo_hbm.at[pl.ds(base, chunk)], fo_v)
        pltpu.async_copy(bb_hbm.at[fb_v], gb_v, sem).wait()
        pltpu.async_copy(bo_hbm.at[fb_v], go_v, sem).wait()

        for t in range(chunk // L):
            off = t * L
            idx = fb_v[pl.ds(off, L)]
            f_ok = fo_v[pl.ds(off, L)]
            g_best = gb_v[pl.ds(off, L)]
            g_ok = go_v[pl.ds(off, L)]
            row = lax.iota(jnp.int32, L) + (base + off)
            ok = (f_ok > 0) & (g_ok > 0) & (g_best == row)
            res = jnp.where(ok, idx, -1)
            oi_v[pl.ds(off, L)] = res
            os_v[pl.ds(off, L)] = jnp.where(res > 0, 1, 0).astype(jnp.int32)

        pltpu.sync_copy(oi_v, idx_hbm.at[pl.ds(base, chunk)])
        pltpu.sync_copy(os_v, ms_hbm.at[pl.ds(base, chunk)])

    return body(fwd_best, fwd_ok, bck_best, bck_ok)


def kernel(descriptors0, descriptors1, keypoints0, keypoints1):
    d0 = descriptors0[0]
    d1 = descriptors1[0]
    N = d0.shape[1]
    M = d1.shape[1]
    fwd_best, fwd_ok, bck_best, bck_ok = _run_top2(d0, d1)
    indices0, mscores0 = _mutual_match_sc(fwd_best, fwd_ok, bck_best, bck_ok)
    indices0 = indices0[None, :]
    mscores0 = mscores0[None, :]
    matches1 = jnp.full((1, M), -1, dtype=jnp.int32)
    mscores1 = jnp.zeros((1, M), dtype=keypoints1.dtype)
    return (indices0, matches1, mscores0, mscores1)


# speedup vs baseline: 137.4842x; 1.2226x over previous
import functools

import jax
import jax.numpy as jnp
from jax import lax
from jax.experimental import pallas as pl
from jax.experimental.pallas import tpu as pltpu
from jax.experimental.pallas import tpu_sc as plsc

SQRT_2 = 1.414213

_BN = 512
_BM = 8192


def _top2_body(d0_ref, d1_ref, fv1, fi1, fv2, fok, bv1, bi1, bv2, bok):
    i = pl.program_id(0)
    j = pl.program_id(1)
    s = lax.dot_general(
        d0_ref[...], d1_ref[...],
        dimension_numbers=(((0,), (0,)), ((), ())),
        preferred_element_type=jnp.float32,
    )

    col_iota = lax.broadcasted_iota(jnp.int32, s.shape, 1)
    row_iota = lax.broadcasted_iota(jnp.int32, s.shape, 0)

    w1 = jnp.max(s, axis=1, keepdims=True)
    eq = s == w1
    a1 = jnp.min(jnp.where(eq, col_iota, _BM), axis=1, keepdims=True) + j * _BM
    cnt = jnp.sum(eq.astype(jnp.int32), axis=1, keepdims=True)
    w2 = jnp.max(jnp.where(eq, -jnp.inf, s), axis=1, keepdims=True)
    w2 = jnp.where(cnt > 1, w1, w2)

    rs = pl.ds(i * _BN, _BN)
    first = j == 0
    cv1 = jnp.where(first, -3.0, fv1[rs, :])
    ci1 = jnp.where(first, 0, fi1[rs, :])
    cv2 = jnp.where(first, -3.0, fv2[rs, :])
    take = w1 > cv1
    m1 = jnp.where(take, w1, cv1)
    mi = jnp.where(take, a1, ci1)
    m2 = jnp.maximum(jnp.minimum(cv1, w1), jnp.maximum(cv2, w2))
    fv1[rs, :] = m1
    fi1[rs, :] = mi
    fv2[rs, :] = m2

    @pl.when(j == pl.num_programs(1) - 1)
    def _fwd_epilogue():
        dist1 = SQRT_2 * jnp.sqrt(jnp.maximum(1.0 - m1, 1e-6))
        dist2 = SQRT_2 * jnp.sqrt(jnp.maximum(1.0 - m2, 1e-6))
        fok[rs, :] = (dist1 / dist2 < 1.0).astype(jnp.int32)

    u1 = jnp.max(s, axis=0, keepdims=True)
    eqb = s == u1
    b1 = jnp.min(jnp.where(eqb, row_iota, _BN), axis=0, keepdims=True) + i * _BN
    cntb = jnp.sum(eqb.astype(jnp.int32), axis=0, keepdims=True)
    u2 = jnp.max(jnp.where(eqb, -jnp.inf, s), axis=0, keepdims=True)
    u2 = jnp.where(cntb > 1, u1, u2)

    firstb = i == 0
    dv1 = jnp.where(firstb, -3.0, bv1[pl.ds(j, 1), :])
    di1 = jnp.where(firstb, 0, bi1[pl.ds(j, 1), :])
    dv2 = jnp.where(firstb, -3.0, bv2[pl.ds(j, 1), :])
    takeb = u1 > dv1
    n1 = jnp.where(takeb, u1, dv1)
    ni = jnp.where(takeb, b1, di1)
    n2 = jnp.maximum(jnp.minimum(dv1, u1), jnp.maximum(dv2, u2))
    bv1[pl.ds(j, 1), :] = n1
    bi1[pl.ds(j, 1), :] = ni
    bv2[pl.ds(j, 1), :] = n2

    @pl.when(i == pl.num_programs(0) - 1)
    def _bck_epilogue():
        bd1 = SQRT_2 * jnp.sqrt(jnp.maximum(1.0 - n1, 1e-6))
        bd2 = SQRT_2 * jnp.sqrt(jnp.maximum(1.0 - n2, 1e-6))
        bok[pl.ds(j, 1), :] = (bd1 / bd2 < 1.0).astype(jnp.int32)


def _run_top2(d0, d1):
    F, N = d0.shape
    _, M = d1.shape
    I, J = N // _BN, M // _BM
    grid = (I, J)
    blk_out = lambda rows, cols: pl.BlockSpec((rows, cols), lambda i, j: (0, 0))
    out_shapes = [
        jax.ShapeDtypeStruct((N, 1), jnp.float32),
        jax.ShapeDtypeStruct((N, 1), jnp.int32),
        jax.ShapeDtypeStruct((N, 1), jnp.float32),
        jax.ShapeDtypeStruct((N, 1), jnp.int32),
        jax.ShapeDtypeStruct((J, _BM), jnp.float32),
        jax.ShapeDtypeStruct((J, _BM), jnp.int32),
        jax.ShapeDtypeStruct((J, _BM), jnp.float32),
        jax.ShapeDtypeStruct((J, _BM), jnp.int32),
    ]
    out_specs = [
        blk_out(N, 1), blk_out(N, 1), blk_out(N, 1), blk_out(N, 1),
        blk_out(J, _BM), blk_out(J, _BM), blk_out(J, _BM), blk_out(J, _BM),
    ]
    outs = pl.pallas_call(
        _top2_body,
        grid=grid,
        in_specs=[
            pl.BlockSpec((F, _BN), lambda i, j: (0, i)),
            pl.BlockSpec((F, _BM), lambda i, j: (0, j)),
        ],
        out_specs=out_specs,
        out_shape=out_shapes,
        compiler_params=pltpu.CompilerParams(
            dimension_semantics=("arbitrary", "arbitrary"),
        ),
    )(d0, d1)
    _, fi1, _, fok, _, bi1, _, bok = outs
    return (fi1.reshape(N), fok.reshape(N), bi1.reshape(M), bok.reshape(M))


def _mutual_match_sc(fwd_best, fwd_ok, bck_best, bck_ok):
    N = fwd_best.shape[0]
    M = bck_best.shape[0]
    info = plsc.get_sparse_core_info()
    NC, NS, L = info.num_cores, info.num_subcores, info.num_lanes
    NW = NC * NS
    chunk = N // NW
    mesh = plsc.VectorSubcoreMesh(core_axis_name="c", subcore_axis_name="s")

    @functools.partial(
        pl.kernel,
        mesh=mesh,
        out_type=[
            jax.ShapeDtypeStruct((N,), jnp.int32),
            jax.ShapeDtypeStruct((N,), jnp.int32),
        ],
        scratch_types=[
            pltpu.VMEM((chunk,), jnp.int32),
            pltpu.VMEM((chunk,), jnp.int32),
            pltpu.VMEM((chunk,), jnp.int32),
            pltpu.VMEM((chunk,), jnp.int32),
            pltpu.VMEM((chunk,), jnp.int32),
            pltpu.VMEM((chunk,), jnp.int32),
            pltpu.SemaphoreType.DMA,
        ],
    )
    def body(fb_hbm, fo_hbm, bb_hbm, bo_hbm, idx_hbm, ms_hbm,
             fb_v, fo_v, gb_v, go_v, oi_v, os_v, sem):
        wid = lax.axis_index("s") * NC + lax.axis_index("c")
        base = wid * chunk
        pltpu.sync_copy(fb_hbm.at[pl.ds(base, chunk)], fb_v)
        pltpu.sync_copy(fo_hbm.at[pl.ds(base, chunk)], fo_v)
        pltpu.async_copy(bb_hbm.at[fb_v], gb_v, sem).wait()
        pltpu.async_copy(bo_hbm.at[fb_v], go_v, sem).wait()

        for t in range(chunk // L):
            off = t * L
            idx = fb_v[pl.ds(off, L)]
            f_ok = fo_v[pl.ds(off, L)]
            g_best = gb_v[pl.ds(off, L)]
            g_ok = go_v[pl.ds(off, L)]
            row = lax.iota(jnp.int32, L) + (base + off)
            ok = (f_ok > 0) & (g_ok > 0) & (g_best == row)
            res = jnp.where(ok, idx, -1)
            oi_v[pl.ds(off, L)] = res
            os_v[pl.ds(off, L)] = jnp.where(res > 0, 1, 0).astype(jnp.int32)

        pltpu.sync_copy(oi_v, idx_hbm.at[pl.ds(base, chunk)])
        pltpu.sync_copy(os_v, ms_hbm.at[pl.ds(base, chunk)])

    return body(fwd_best, fwd_ok, bck_best, bck_ok)


def kernel(descriptors0, descriptors1, keypoints0, keypoints1):
    d0 = descriptors0[0]
    d1 = descriptors1[0]
    N = d0.shape[1]
    M = d1.shape[1]
    fwd_best, fwd_ok, bck_best, bck_ok = _run_top2(d0, d1)
    indices0, mscores0 = _mutual_match_sc(fwd_best, fwd_ok, bck_best, bck_ok)
    indices0 = indices0[None, :]
    mscores0 = mscores0[None, :]
    matches1 = jnp.full((1, M), -1, dtype=jnp.int32)
    mscores1 = jnp.zeros((1, M), dtype=keypoints1.dtype)
    return (indices0, matches1, mscores0, mscores1)


# speedup vs baseline: 146.3750x; 1.0647x over previous
import functools

import jax
import jax.numpy as jnp
from jax import lax
from jax.experimental import pallas as pl
from jax.experimental.pallas import tpu as pltpu
from jax.experimental.pallas import tpu_sc as plsc

SQRT_2 = 1.414213

_BN = 512
_BM = 4096


def _top2_body(d0_ref, d1_ref, fv1, fi1, fv2, fok, bv1, bi1, bv2, bok):
    i = pl.program_id(0)
    j = pl.program_id(1)
    s = lax.dot_general(
        d0_ref[...], d1_ref[...],
        dimension_numbers=(((0,), (0,)), ((), ())),
        preferred_element_type=jnp.float32,
    )

    w1 = jnp.max(s, axis=1, keepdims=True)
    eq = s == w1
    eqf = eq.astype(jnp.float32)
    iota_f = lax.broadcasted_iota(jnp.int32, (_BM, 1), 0)
    io_f = jnp.concatenate(
        [(iota_f // 32).astype(jnp.float32),
         (iota_f % 32).astype(jnp.float32),
         jnp.ones((_BM, 1), jnp.float32)], axis=1)
    r = lax.dot_general(
        eqf, io_f, dimension_numbers=(((1,), (0,)), ((), ())),
        preferred_element_type=jnp.float32)
    idx_f = r[:, 0:1] * 32.0 + r[:, 1:2]
    a1 = jnp.minimum(idx_f.astype(jnp.int32), _BM - 1) + j * _BM
    cnt = r[:, 2:3]
    w2 = jnp.max(jnp.where(eq, -jnp.inf, s), axis=1, keepdims=True)
    w2 = jnp.where(cnt > 1.5, w1, w2)

    rs = pl.ds(i * _BN, _BN)
    first = j == 0
    cv1 = jnp.where(first, -3.0, fv1[rs, :])
    ci1 = jnp.where(first, 0, fi1[rs, :])
    cv2 = jnp.where(first, -3.0, fv2[rs, :])
    take = w1 > cv1
    m1 = jnp.where(take, w1, cv1)
    mi = jnp.where(take, a1, ci1)
    m2 = jnp.maximum(jnp.minimum(cv1, w1), jnp.maximum(cv2, w2))
    fv1[rs, :] = m1
    fi1[rs, :] = mi
    fv2[rs, :] = m2

    @pl.when(j == pl.num_programs(1) - 1)
    def _fwd_epilogue():
        dist1 = SQRT_2 * jnp.sqrt(jnp.maximum(1.0 - m1, 1e-6))
        dist2 = SQRT_2 * jnp.sqrt(jnp.maximum(1.0 - m2, 1e-6))
        fok[rs, :] = (dist1 / dist2 < 1.0).astype(jnp.int32)

    u1 = jnp.max(s, axis=0, keepdims=True)
    eqb = s == u1
    eqbf = eqb.astype(jnp.float32)
    iota_b = lax.broadcasted_iota(jnp.int32, (1, _BN), 1)
    io_b = jnp.concatenate(
        [(iota_b // 32).astype(jnp.float32),
         (iota_b % 32).astype(jnp.float32),
         jnp.ones((1, _BN), jnp.float32)], axis=0)
    rb = lax.dot_general(
        io_b, eqbf, dimension_numbers=(((1,), (0,)), ((), ())),
        preferred_element_type=jnp.float32)
    idx_b = rb[0:1, :] * 32.0 + rb[1:2, :]
    b1 = idx_b.astype(jnp.int32) + i * _BN
    cntb = rb[2:3, :]
    u2 = jnp.max(jnp.where(eqb, -jnp.inf, s), axis=0, keepdims=True)
    u2 = jnp.where(cntb > 1.5, u1, u2)

    firstb = i == 0
    dv1 = jnp.where(firstb, -3.0, bv1[pl.ds(j, 1), :])
    di1 = jnp.where(firstb, 0, bi1[pl.ds(j, 1), :])
    dv2 = jnp.where(firstb, -3.0, bv2[pl.ds(j, 1), :])
    takeb = u1 > dv1
    n1 = jnp.where(takeb, u1, dv1)
    ni = jnp.where(takeb, b1, di1)
    n2 = jnp.maximum(jnp.minimum(dv1, u1), jnp.maximum(dv2, u2))
    bv1[pl.ds(j, 1), :] = n1
    bi1[pl.ds(j, 1), :] = ni
    bv2[pl.ds(j, 1), :] = n2

    @pl.when(i == pl.num_programs(0) - 1)
    def _bck_epilogue():
        bd1 = SQRT_2 * jnp.sqrt(jnp.maximum(1.0 - n1, 1e-6))
        bd2 = SQRT_2 * jnp.sqrt(jnp.maximum(1.0 - n2, 1e-6))
        bok[pl.ds(j, 1), :] = (bd1 / bd2 < 1.0).astype(jnp.int32)


def _run_top2(d0, d1):
    F, N = d0.shape
    _, M = d1.shape
    I, J = N // _BN, M // _BM
    grid = (I, J)
    blk_out = lambda rows, cols: pl.BlockSpec((rows, cols), lambda i, j: (0, 0))
    out_shapes = [
        jax.ShapeDtypeStruct((N, 1), jnp.float32),
        jax.ShapeDtypeStruct((N, 1), jnp.int32),
        jax.ShapeDtypeStruct((N, 1), jnp.float32),
        jax.ShapeDtypeStruct((N, 1), jnp.int32),
        jax.ShapeDtypeStruct((J, _BM), jnp.float32),
        jax.ShapeDtypeStruct((J, _BM), jnp.int32),
        jax.ShapeDtypeStruct((J, _BM), jnp.float32),
        jax.ShapeDtypeStruct((J, _BM), jnp.int32),
    ]
    out_specs = [
        blk_out(N, 1), blk_out(N, 1), blk_out(N, 1), blk_out(N, 1),
        blk_out(J, _BM), blk_out(J, _BM), blk_out(J, _BM), blk_out(J, _BM),
    ]
    outs = pl.pallas_call(
        _top2_body,
        grid=grid,
        in_specs=[
            pl.BlockSpec((F, _BN), lambda i, j: (0, i)),
            pl.BlockSpec((F, _BM), lambda i, j: (0, j)),
        ],
        out_specs=out_specs,
        out_shape=out_shapes,
        compiler_params=pltpu.CompilerParams(
            dimension_semantics=("arbitrary", "arbitrary"),
        ),
    )(d0, d1)
    _, fi1, _, fok, _, bi1, _, bok = outs
    return (fi1.reshape(N), fok.reshape(N), bi1.reshape(M), bok.reshape(M))


def _mutual_match_sc(fwd_best, fwd_ok, bck_best, bck_ok):
    N = fwd_best.shape[0]
    M = bck_best.shape[0]
    info = plsc.get_sparse_core_info()
    NC, NS, L = info.num_cores, info.num_subcores, info.num_lanes
    NW = NC * NS
    chunk = N // NW
    mesh = plsc.VectorSubcoreMesh(core_axis_name="c", subcore_axis_name="s")

    @functools.partial(
        pl.kernel,
        mesh=mesh,
        out_type=[
            jax.ShapeDtypeStruct((N,), jnp.int32),
            jax.ShapeDtypeStruct((N,), jnp.int32),
        ],
        scratch_types=[
            pltpu.VMEM((chunk,), jnp.int32),
            pltpu.VMEM((chunk,), jnp.int32),
            pltpu.VMEM((chunk,), jnp.int32),
            pltpu.VMEM((chunk,), jnp.int32),
            pltpu.VMEM((chunk,), jnp.int32),
            pltpu.VMEM((chunk,), jnp.int32),
            pltpu.SemaphoreType.DMA,
        ],
    )
    def body(fb_hbm, fo_hbm, bb_hbm, bo_hbm, idx_hbm, ms_hbm,
             fb_v, fo_v, gb_v, go_v, oi_v, os_v, sem):
        wid = lax.axis_index("s") * NC + lax.axis_index("c")
        base = wid * chunk
        pltpu.sync_copy(fb_hbm.at[pl.ds(base, chunk)], fb_v)
        pltpu.sync_copy(fo_hbm.at[pl.ds(base, chunk)], fo_v)
        pltpu.async_copy(bb_hbm.at[fb_v], gb_v, sem).wait()
        pltpu.async_copy(bo_hbm.at[fb_v], go_v, sem).wait()

        for t in range(chunk // L):
            off = t * L
            idx = fb_v[pl.ds(off, L)]
            f_ok = fo_v[pl.ds(off, L)]
            g_best = gb_v[pl.ds(off, L)]
            g_ok = go_v[pl.ds(off, L)]
            row = lax.iota(jnp.int32, L) + (base + off)
            ok = (f_ok > 0) & (g_ok > 0) & (g_best == row)
            res = jnp.where(ok, idx, -1)
            oi_v[pl.ds(off, L)] = res
            os_v[pl.ds(off, L)] = jnp.where(res > 0, 1, 0).astype(jnp.int32)

        pltpu.sync_copy(oi_v, idx_hbm.at[pl.ds(base, chunk)])
        pltpu.sync_copy(os_v, ms_hbm.at[pl.ds(base, chunk)])

    return body(fwd_best, fwd_ok, bck_best, bck_ok)


def kernel(descriptors0, descriptors1, keypoints0, keypoints1):
    d0 = descriptors0[0]
    d1 = descriptors1[0]
    N = d0.shape[1]
    M = d1.shape[1]
    fwd_best, fwd_ok, bck_best, bck_ok = _run_top2(d0, d1)
    indices0, mscores0 = _mutual_match_sc(fwd_best, fwd_ok, bck_best, bck_ok)
    indices0 = indices0[None, :]
    mscores0 = mscores0[None, :]
    matches1 = jnp.full((1, M), -1, dtype=jnp.int32)
    mscores1 = jnp.zeros((1, M), dtype=keypoints1.dtype)
    return (indices0, matches1, mscores0, mscores1)


# speedup vs baseline: 193.2517x; 1.3203x over previous
import functools

import jax
import jax.numpy as jnp
from jax import lax
from jax.experimental import pallas as pl
from jax.experimental.pallas import tpu as pltpu
from jax.experimental.pallas import tpu_sc as plsc

SQRT_2 = 1.414213

_BN = 512
_BM = 8192


def _top2_body(d0_ref, d1_ref, fv1, fi1, fv2, fok, bv1, bi1, bv2, bok):
    i = pl.program_id(0)
    j = pl.program_id(1)
    s = lax.dot_general(
        d0_ref[...], d1_ref[...],
        dimension_numbers=(((0,), (0,)), ((), ())),
        preferred_element_type=jnp.float32,
    )

    w1 = jnp.max(s, axis=1, keepdims=True)
    eq = s == w1
    eqf = eq.astype(jnp.float32)
    iota_f = lax.broadcasted_iota(jnp.int32, (_BM, 1), 0)
    io_f = jnp.concatenate(
        [(iota_f // 32).astype(jnp.float32),
         (iota_f % 32).astype(jnp.float32),
         jnp.ones((_BM, 1), jnp.float32)], axis=1)
    r = lax.dot_general(
        eqf, io_f, dimension_numbers=(((1,), (0,)), ((), ())),
        preferred_element_type=jnp.float32)
    idx_f = r[:, 0:1] * 32.0 + r[:, 1:2]
    cnt = r[:, 2:3]
    w2 = jnp.max(jnp.where(eq, -jnp.inf, s), axis=1, keepdims=True)
    w2 = jnp.where(cnt > 1.5, w1, w2)

    w1t = lax.transpose(w1, (1, 0))
    w2t = lax.transpose(w2, (1, 0))
    idx_t = lax.transpose(idx_f, (1, 0))
    a1 = jnp.minimum(idx_t.astype(jnp.int32), _BM - 1) + j * _BM

    ri = pl.ds(i, 1)
    first = j == 0
    cv1 = jnp.where(first, -3.0, fv1[ri, :])
    ci1 = jnp.where(first, 0, fi1[ri, :])
    cv2 = jnp.where(first, -3.0, fv2[ri, :])
    take = w1t > cv1
    m1 = jnp.where(take, w1t, cv1)
    mi = jnp.where(take, a1, ci1)
    m2 = jnp.maximum(jnp.minimum(cv1, w1t), jnp.maximum(cv2, w2t))
    fv1[ri, :] = m1
    fi1[ri, :] = mi
    fv2[ri, :] = m2

    @pl.when(j == pl.num_programs(1) - 1)
    def _fwd_epilogue():
        dist1 = SQRT_2 * jnp.sqrt(jnp.maximum(1.0 - m1, 1e-6))
        dist2 = SQRT_2 * jnp.sqrt(jnp.maximum(1.0 - m2, 1e-6))
        fok[ri, :] = (dist1 / dist2 < 1.0).astype(jnp.int32)

    u1 = jnp.max(s, axis=0, keepdims=True)
    eqb = s == u1
    eqbf = eqb.astype(jnp.float32)
    iota_b = lax.broadcasted_iota(jnp.int32, (1, _BN), 1)
    io_b = jnp.concatenate(
        [(iota_b // 32).astype(jnp.float32),
         (iota_b % 32).astype(jnp.float32),
         jnp.ones((1, _BN), jnp.float32)], axis=0)
    rb = lax.dot_general(
        io_b, eqbf, dimension_numbers=(((1,), (0,)), ((), ())),
        preferred_element_type=jnp.float32)
    idx_b = rb[0:1, :] * 32.0 + rb[1:2, :]
    b1 = idx_b.astype(jnp.int32) + i * _BN
    cntb = rb[2:3, :]
    u2 = jnp.max(jnp.where(eqb, -jnp.inf, s), axis=0, keepdims=True)
    u2 = jnp.where(cntb > 1.5, u1, u2)

    firstb = i == 0
    dv1 = jnp.where(firstb, -3.0, bv1[pl.ds(j, 1), :])
    di1 = jnp.where(firstb, 0, bi1[pl.ds(j, 1), :])
    dv2 = jnp.where(firstb, -3.0, bv2[pl.ds(j, 1), :])
    takeb = u1 > dv1
    n1 = jnp.where(takeb, u1, dv1)
    ni = jnp.where(takeb, b1, di1)
    n2 = jnp.maximum(jnp.minimum(dv1, u1), jnp.maximum(dv2, u2))
    bv1[pl.ds(j, 1), :] = n1
    bi1[pl.ds(j, 1), :] = ni
    bv2[pl.ds(j, 1), :] = n2

    @pl.when(i == pl.num_programs(0) - 1)
    def _bck_epilogue():
        bd1 = SQRT_2 * jnp.sqrt(jnp.maximum(1.0 - n1, 1e-6))
        bd2 = SQRT_2 * jnp.sqrt(jnp.maximum(1.0 - n2, 1e-6))
        bok[pl.ds(j, 1), :] = (bd1 / bd2 < 1.0).astype(jnp.int32)


def _run_top2(d0, d1):
    F, N = d0.shape
    _, M = d1.shape
    I, J = N // _BN, M // _BM
    grid = (I, J)
    blk_out = lambda rows, cols: pl.BlockSpec((rows, cols), lambda i, j: (0, 0))
    out_shapes = [
        jax.ShapeDtypeStruct((I, _BN), jnp.float32),
        jax.ShapeDtypeStruct((I, _BN), jnp.int32),
        jax.ShapeDtypeStruct((I, _BN), jnp.float32),
        jax.ShapeDtypeStruct((I, _BN), jnp.int32),
        jax.ShapeDtypeStruct((J, _BM), jnp.float32),
        jax.ShapeDtypeStruct((J, _BM), jnp.int32),
        jax.ShapeDtypeStruct((J, _BM), jnp.float32),
        jax.ShapeDtypeStruct((J, _BM), jnp.int32),
    ]
    out_specs = [
        blk_out(I, _BN), blk_out(I, _BN), blk_out(I, _BN), blk_out(I, _BN),
        blk_out(J, _BM), blk_out(J, _BM), blk_out(J, _BM), blk_out(J, _BM),
    ]
    outs = pl.pallas_call(
        _top2_body,
        grid=grid,
        in_specs=[
            pl.BlockSpec((F, _BN), lambda i, j: (0, i)),
            pl.BlockSpec((F, _BM), lambda i, j: (0, j)),
        ],
        out_specs=out_specs,
        out_shape=out_shapes,
        compiler_params=pltpu.CompilerParams(
            dimension_semantics=("arbitrary", "arbitrary"),
        ),
    )(d0, d1)
    _, fi1, _, fok, _, bi1, _, bok = outs
    return (fi1.reshape(N), fok.reshape(N), bi1.reshape(M), bok.reshape(M))


def _mutual_match_sc(fwd_best, fwd_ok, bck_best, bck_ok):
    N = fwd_best.shape[0]
    M = bck_best.shape[0]
    info = plsc.get_sparse_core_info()
    NC, NS, L = info.num_cores, info.num_subcores, info.num_lanes
    NW = NC * NS
    chunk = N // NW
    mesh = plsc.VectorSubcoreMesh(core_axis_name="c", subcore_axis_name="s")

    @functools.partial(
        pl.kernel,
        mesh=mesh,
        out_type=[
            jax.ShapeDtypeStruct((N,), jnp.int32),
            jax.ShapeDtypeStruct((N,), jnp.int32),
        ],
        scratch_types=[
            pltpu.VMEM((chunk,), jnp.int32),
            pltpu.VMEM((chunk,), jnp.int32),
            pltpu.VMEM((chunk,), jnp.int32),
            pltpu.VMEM((chunk,), jnp.int32),
            pltpu.VMEM((chunk,), jnp.int32),
            pltpu.VMEM((chunk,), jnp.int32),
            pltpu.SemaphoreType.DMA,
        ],
    )
    def body(fb_hbm, fo_hbm, bb_hbm, bo_hbm, idx_hbm, ms_hbm,
             fb_v, fo_v, gb_v, go_v, oi_v, os_v, sem):
        wid = lax.axis_index("s") * NC + lax.axis_index("c")
        base = wid * chunk
        pltpu.sync_copy(fb_hbm.at[pl.ds(base, chunk)], fb_v)
        pltpu.sync_copy(fo_hbm.at[pl.ds(base, chunk)], fo_v)
        pltpu.async_copy(bb_hbm.at[fb_v], gb_v, sem).wait()
        pltpu.async_copy(bo_hbm.at[fb_v], go_v, sem).wait()

        for t in range(chunk // L):
            off = t * L
            idx = fb_v[pl.ds(off, L)]
            f_ok = fo_v[pl.ds(off, L)]
            g_best = gb_v[pl.ds(off, L)]
            g_ok = go_v[pl.ds(off, L)]
            row = lax.iota(jnp.int32, L) + (base + off)
            ok = (f_ok > 0) & (g_ok > 0) & (g_best == row)
            res = jnp.where(ok, idx, -1)
            oi_v[pl.ds(off, L)] = res
            os_v[pl.ds(off, L)] = jnp.where(res > 0, 1, 0).astype(jnp.int32)

        pltpu.sync_copy(oi_v, idx_hbm.at[pl.ds(base, chunk)])
        pltpu.sync_copy(os_v, ms_hbm.at[pl.ds(base, chunk)])

    return body(fwd_best, fwd_ok, bck_best, bck_ok)


def kernel(descriptors0, descriptors1, keypoints0, keypoints1):
    d0 = descriptors0[0]
    d1 = descriptors1[0]
    N = d0.shape[1]
    M = d1.shape[1]
    fwd_best, fwd_ok, bck_best, bck_ok = _run_top2(d0, d1)
    indices0, mscores0 = _mutual_match_sc(fwd_best, fwd_ok, bck_best, bck_ok)
    indices0 = indices0[None, :]
    mscores0 = mscores0[None, :]
    matches1 = jnp.full((1, M), -1, dtype=jnp.int32)
    mscores1 = jnp.zeros((1, M), dtype=keypoints1.dtype)
    return (indices0, matches1, mscores0, mscores1)
